# baseline (speedup 1.0000x reference)
import functools

import jax
import jax.numpy as jnp
from jax import lax
from jax.experimental import pallas as pl
from jax.experimental.pallas import tpu as pltpu
from jax.experimental.pallas import tpu_sc as plsc

B = 16384
D = 64
K = 20
V = 1000000

_info = plsc.get_sparse_core_info()
NC, NS = _info.num_cores, _info.num_subcores
NW = NC * NS
BPW = B // NW
C = 32
NCHUNK = BPW // C
SB = (1 + K) * BPW

CB = 2048
NPAIR = (V + 2 * CB - 1) // (2 * CB)
VP = NPAIR * 2 * CB


def _tp_body(a_ref, b_ref, o_ref):
    o_ref[...] = jnp.concatenate([a_ref[...].T, b_ref[...].T], axis=1)


def _transpose_tc(embT):
    return pl.pallas_call(
        _tp_body,
        grid=(NPAIR,),
        in_specs=[pl.BlockSpec((D, CB), lambda i: (0, 2 * i)),
                  pl.BlockSpec(
                      (D, CB),
                      lambda i: (0, jnp.minimum(2 * i + 1, (V - 1) // CB)))],
        out_specs=pl.BlockSpec((CB, 2 * D), lambda i: (i, 0)),
        out_shape=jax.ShapeDtypeStruct((NPAIR * CB, 2 * D), jnp.float32),
    )(embT, embT)


def _sc_scores(centers, contexts, neg_flat, iemb, oemb):
    mesh = plsc.VectorSubcoreMesh(core_axis_name="c", subcore_axis_name="s")

    @functools.partial(
        pl.kernel,
        mesh=mesh,
        out_type=jax.ShapeDtypeStruct((NW * SB,), jnp.float32),
        compiler_params=pltpu.CompilerParams(
            needs_layout_passes=False, use_tc_tiling_on_sc=False),
        scratch_types=[
            pltpu.VMEM((BPW,), jnp.int32),
            pltpu.VMEM((BPW,), jnp.int32),
            pltpu.VMEM((BPW * K,), jnp.int32),
            pltpu.VMEM((C, D), jnp.float32),
            pltpu.VMEM((C, D), jnp.float32),
            pltpu.VMEM((C, D), jnp.float32),
            pltpu.VMEM((C, D), jnp.float32),
            pltpu.VMEM((C * K, D), jnp.float32),
            pltpu.VMEM((C * K, D), jnp.float32),
            pltpu.VMEM((SB,), jnp.float32),
            pltpu.SemaphoreType.DMA,
            pltpu.SemaphoreType.DMA,
        ],
    )
    def body(cen_h, ctx_h, neg_h, iemb_h, oemb_h, out_h,
             cidxa, xidxa, nidxa, cb0, cb1, xb0, xb1, nb0, nb1,
             sbuf, sem0, sem1):
        wid = lax.axis_index("s") * NC + lax.axis_index("c")
        base = wid * BPW
        lanes = lax.iota(jnp.int32, 16)
        pltpu.sync_copy(cen_h.at[pl.ds(base, BPW)], cidxa)
        pltpu.sync_copy(ctx_h.at[pl.ds(base, BPW)], xidxa)
        pltpu.sync_copy(neg_h.at[pl.ds(base * K, BPW * K)], nidxa)

        def remap(buf, n):
            def step(i, _):
                sl = pl.ds(i * 16, 16)
                ids = buf[sl]
                buf[sl] = ((ids & jnp.int32(~4095))
                           + ((ids & 2047) << 1) + ((ids >> 11) & 1))
                return 0
            lax.fori_loop(0, n // 16, step, 0)

        remap(cidxa, BPW)
        remap(xidxa, BPW)
        remap(nidxa, BPW * K)
        bufs = ((cb0, xb0, nb0, sem0), (cb1, xb1, nb1, sem1))

        def descs(t, p):
            cb, xb, nb, sem = bufs[p]
            off = t * C
            return [
                pltpu.make_async_copy(
                    iemb_h.at[cidxa.at[pl.ds(off, C)]], cb, sem),
                pltpu.make_async_copy(
                    oemb_h.at[xidxa.at[pl.ds(off, C)]], xb, sem),
                pltpu.make_async_copy(
                    oemb_h.at[nidxa.at[pl.ds(off * K, C * K)]], nb, sem),
            ]

        def compute(t, p):
            cb, xb, nb, _ = bufs[p]
            for g in range(C // 16):
                rows = g * 16 + lanes
                rows_k = rows * K

                def dstep(dd, accs):
                    new = list(accs)
                    for u in range(4):
                        col = jnp.broadcast_to(
                            dd * 4 + u, (16,)).astype(jnp.int32)
                        ccol = plsc.load_gather(cb, [rows, col])
                        xcol = plsc.load_gather(xb, [rows, col])
                        new[0] = new[0] + ccol * xcol
                        for k in range(K):
                            ncol = plsc.load_gather(nb, [rows_k + k, col])
                            new[k + 1] = new[k + 1] + ccol * ncol
                    return tuple(new)

                accs = lax.fori_loop(
                    0, D // 4, dstep,
                    tuple(jnp.zeros((16,), jnp.float32)
                          for _ in range(1 + K)))
                eoff = t * C + g * 16
                sbuf[pl.ds(eoff, 16)] = -accs[0]
                for k in range(K):
                    sbuf[pl.ds((k + 1) * BPW + eoff, 16)] = accs[k + 1]

        for cp in descs(0, 0):
            cp.start()

        def pair(tt, _):
            for p in range(2):
                t = tt * 2 + p

                @pl.when(t + 1 < NCHUNK)
                def _():
                    for cp in descs(t + 1, 1 - p):
                        cp.start()

                for cp in descs(t, p):
                    cp.wait()
                compute(t, p)
            return 0

        lax.fori_loop(0, NCHUNK // 2, pair, 0)
        pltpu.sync_copy(sbuf, out_h.at[pl.ds(wid * SB, SB)])

    return body(centers, contexts, neg_flat, iemb, oemb)


def _loss_tc(scores):
    def body(s_ref, o_ref):
        x = s_ref[...]
        sp = jnp.maximum(x, 0.0) + jnp.log1p(jnp.exp(-jnp.abs(x)))
        o_ref[0, 0] = jnp.sum(sp) * (1.0 / B)

    return pl.pallas_call(
        body,
        out_shape=jax.ShapeDtypeStruct((1, 1), jnp.float32),
        out_specs=pl.BlockSpec(memory_space=pltpu.SMEM),
    )(scores)


def kernel(input_emb, output_emb, centers, contexts, negatives):
    iemb = _transpose_tc(input_emb.T).reshape(VP, D)
    oemb = _transpose_tc(output_emb.T).reshape(VP, D)
    neg_flat = negatives.astype(jnp.int32).reshape(B * K)
    scores = _sc_scores(centers.astype(jnp.int32), contexts.astype(jnp.int32),
                        neg_flat, iemb, oemb)
    loss = _loss_tc(scores.reshape(NW * (1 + K), BPW))
    return loss[0, 0]


# speedup vs baseline: 1.0283x; 1.0283x over previous
import functools

import jax
import jax.numpy as jnp
from jax import lax
from jax.experimental import pallas as pl
from jax.experimental.pallas import tpu as pltpu
from jax.experimental.pallas import tpu_sc as plsc

B = 16384
D = 64
K = 20
V = 1000000

_info = plsc.get_sparse_core_info()
NC, NS = _info.num_cores, _info.num_subcores
NW = NC * NS
BPW = B // NW
C = 32
NCHUNK = BPW // C
SB = (1 + K) * BPW

CB = 2048
NPAIR = (V + 2 * CB - 1) // (2 * CB)
VP = NPAIR * 2 * CB


def _tp_body(a_ref, b_ref, o_ref):
    o_ref[...] = jnp.concatenate([a_ref[...].T, b_ref[...].T], axis=1)


def _transpose_tc(embT):
    return pl.pallas_call(
        _tp_body,
        grid=(NPAIR,),
        in_specs=[pl.BlockSpec((D, CB), lambda i: (0, 2 * i)),
                  pl.BlockSpec(
                      (D, CB),
                      lambda i: (0, jnp.minimum(2 * i + 1, (V - 1) // CB)))],
        out_specs=pl.BlockSpec((CB, 2 * D), lambda i: (i, 0)),
        out_shape=jax.ShapeDtypeStruct((NPAIR * CB, 2 * D), jnp.float32),
    )(embT, embT)


def _sc_scores(centers, contexts, neg_flat, iemb, oemb):
    mesh = plsc.VectorSubcoreMesh(core_axis_name="c", subcore_axis_name="s")

    @functools.partial(
        pl.kernel,
        mesh=mesh,
        out_type=jax.ShapeDtypeStruct((NW * SB,), jnp.float32),
        compiler_params=pltpu.CompilerParams(
            needs_layout_passes=False, use_tc_tiling_on_sc=False),
        scratch_types=[
            pltpu.VMEM((BPW,), jnp.int32),
            pltpu.VMEM((BPW,), jnp.int32),
            pltpu.VMEM((BPW * K,), jnp.int32),
            pltpu.VMEM((C, D), jnp.float32),
            pltpu.VMEM((C, D), jnp.float32),
            pltpu.VMEM((C, D), jnp.float32),
            pltpu.VMEM((C, D), jnp.float32),
            pltpu.VMEM((C * K, D), jnp.float32),
            pltpu.VMEM((C * K, D), jnp.float32),
            pltpu.VMEM((SB,), jnp.float32),
            pltpu.SemaphoreType.DMA,
            pltpu.SemaphoreType.DMA,
        ],
    )
    def body(cen_h, ctx_h, neg_h, iemb_h, oemb_h, out_h,
             cidxa, xidxa, nidxa, cb0, cb1, xb0, xb1, nb0, nb1,
             sbuf, sem0, sem1):
        wid = lax.axis_index("s") * NC + lax.axis_index("c")
        base = wid * BPW
        lanes = lax.iota(jnp.int32, 16)
        pltpu.sync_copy(cen_h.at[pl.ds(base, BPW)], cidxa)
        pltpu.sync_copy(ctx_h.at[pl.ds(base, BPW)], xidxa)
        pltpu.sync_copy(neg_h.at[pl.ds(base * K, BPW * K)], nidxa)

        def remap(buf, n):
            def step(i, _):
                sl = pl.ds(i * 16, 16)
                ids = buf[sl]
                buf[sl] = ((ids & jnp.int32(~4095))
                           + ((ids & 2047) << 1) + ((ids >> 11) & 1))
                return 0
            lax.fori_loop(0, n // 16, step, 0)

        remap(cidxa, BPW)
        remap(xidxa, BPW)
        remap(nidxa, BPW * K)
        bufs = ((cb0, xb0, nb0, sem0), (cb1, xb1, nb1, sem1))

        def descs(t, p):
            cb, xb, nb, sem = bufs[p]
            off = t * C
            return [
                pltpu.make_async_copy(
                    iemb_h.at[cidxa.at[pl.ds(off, C)]], cb, sem),
                pltpu.make_async_copy(
                    oemb_h.at[xidxa.at[pl.ds(off, C)]], xb, sem),
                pltpu.make_async_copy(
                    oemb_h.at[nidxa.at[pl.ds(off * K, C * K)]], nb, sem),
            ]

        def compute(t, p):
            cb, xb, nb, _ = bufs[p]
            for g in range(C // 16):
                rows = g * 16 + lanes
                rows_k = rows * K
                eoff = t * C + g * 16

                def dstep_a(dd, accs):
                    new = list(accs)
                    for u in range(4):
                        col = jnp.broadcast_to(
                            dd * 4 + u, (16,)).astype(jnp.int32)
                        ccol = plsc.load_gather(cb, [rows, col])
                        xcol = plsc.load_gather(xb, [rows, col])
                        new[0] = new[0] + ccol * xcol
                        for k in range(K // 2):
                            ncol = plsc.load_gather(nb, [rows_k + k, col])
                            new[k + 1] = new[k + 1] + ccol * ncol
                    return tuple(new)

                accs = lax.fori_loop(
                    0, D // 4, dstep_a,
                    tuple(jnp.zeros((16,), jnp.float32)
                          for _ in range(1 + K // 2)))
                sbuf[pl.ds(eoff, 16)] = -accs[0]
                for k in range(K // 2):
                    sbuf[pl.ds((k + 1) * BPW + eoff, 16)] = accs[k + 1]

                def dstep_b(dd, accs):
                    new = list(accs)
                    for u in range(4):
                        col = jnp.broadcast_to(
                            dd * 4 + u, (16,)).astype(jnp.int32)
                        ccol = plsc.load_gather(cb, [rows, col])
                        for k in range(K // 2):
                            ncol = plsc.load_gather(
                                nb, [rows_k + (K // 2 + k), col])
                            new[k] = new[k] + ccol * ncol
                    return tuple(new)

                accs = lax.fori_loop(
                    0, D // 4, dstep_b,
                    tuple(jnp.zeros((16,), jnp.float32)
                          for _ in range(K // 2)))
                for k in range(K // 2):
                    sbuf[pl.ds((K // 2 + k + 1) * BPW + eoff, 16)] = accs[k]

        for cp in descs(0, 0):
            cp.start()

        def pair(tt, _):
            for p in range(2):
                t = tt * 2 + p

                @pl.when(t + 1 < NCHUNK)
                def _():
                    for cp in descs(t + 1, 1 - p):
                        cp.start()

                for cp in descs(t, p):
                    cp.wait()
                compute(t, p)
            return 0

        lax.fori_loop(0, NCHUNK // 2, pair, 0)
        pltpu.sync_copy(sbuf, out_h.at[pl.ds(wid * SB, SB)])

    return body(centers, contexts, neg_flat, iemb, oemb)


def _loss_tc(scores):
    def body(s_ref, o_ref):
        x = s_ref[...]
        sp = jnp.maximum(x, 0.0) + jnp.log1p(jnp.exp(-jnp.abs(x)))
        o_ref[0, 0] = jnp.sum(sp) * (1.0 / B)

    return pl.pallas_call(
        body,
        out_shape=jax.ShapeDtypeStruct((1, 1), jnp.float32),
        out_specs=pl.BlockSpec(memory_space=pltpu.SMEM),
    )(scores)


def kernel(input_emb, output_emb, centers, contexts, negatives):
    iemb = _transpose_tc(input_emb.T).reshape(VP, D)
    oemb = _transpose_tc(output_emb.T).reshape(VP, D)
    neg_flat = negatives.astype(jnp.int32).reshape(B * K)
    scores = _sc_scores(centers.astype(jnp.int32), contexts.astype(jnp.int32),
                        neg_flat, iemb, oemb)
    loss = _loss_tc(scores.reshape(NW * (1 + K), BPW))
    return loss[0, 0]


# speedup vs baseline: 1.1739x; 1.1415x over previous
import functools

import jax
import jax.numpy as jnp
from jax import lax
from jax.experimental import pallas as pl
from jax.experimental.pallas import tpu as pltpu
from jax.experimental.pallas import tpu_sc as plsc

B = 16384
D = 64
K = 20
V = 1000000

_info = plsc.get_sparse_core_info()
NC, NS = _info.num_cores, _info.num_subcores
NW = NC * NS
BPW = B // NW
C = 32
NCHUNK = BPW // C
SB = (1 + K) * BPW

CB = 4096
LOG2CB = 12
NPAIR = (V + 2 * CB - 1) // (2 * CB)
VP = NPAIR * 2 * CB


def _tp_body(a_ref, b_ref, o_ref):
    o_ref[...] = jnp.concatenate([a_ref[...].T, b_ref[...].T], axis=1)


def _transpose_tc(embT):
    return pl.pallas_call(
        _tp_body,
        grid=(NPAIR,),
        in_specs=[pl.BlockSpec((D, CB), lambda i: (0, 2 * i)),
                  pl.BlockSpec(
                      (D, CB),
                      lambda i: (0, jnp.minimum(2 * i + 1, (V - 1) // CB)))],
        out_specs=pl.BlockSpec((CB, 2 * D), lambda i: (i, 0)),
        out_shape=jax.ShapeDtypeStruct((NPAIR * CB, 2 * D), jnp.float32),
    )(embT, embT)


def _sc_scores(centers, contexts, neg_flat, iemb, oemb):
    mesh = plsc.VectorSubcoreMesh(core_axis_name="c", subcore_axis_name="s")

    @functools.partial(
        pl.kernel,
        mesh=mesh,
        out_type=jax.ShapeDtypeStruct((NW * SB,), jnp.float32),
        compiler_params=pltpu.CompilerParams(
            needs_layout_passes=False, use_tc_tiling_on_sc=False),
        scratch_types=[
            pltpu.VMEM((BPW,), jnp.int32),
            pltpu.VMEM((BPW,), jnp.int32),
            pltpu.VMEM((BPW * K,), jnp.int32),
            pltpu.VMEM((C, D), jnp.float32),
            pltpu.VMEM((C, D), jnp.float32),
            pltpu.VMEM((C, D), jnp.float32),
            pltpu.VMEM((C, D), jnp.float32),
            pltpu.VMEM((C * K, D), jnp.float32),
            pltpu.VMEM((C * K, D), jnp.float32),
            pltpu.VMEM((SB,), jnp.float32),
            pltpu.SemaphoreType.DMA,
            pltpu.SemaphoreType.DMA,
        ],
    )
    def body(cen_h, ctx_h, neg_h, iemb_h, oemb_h, out_h,
             cidxa, xidxa, nidxa, cb0, cb1, xb0, xb1, nb0, nb1,
             sbuf, sem0, sem1):
        wid = lax.axis_index("s") * NC + lax.axis_index("c")
        base = wid * BPW
        lanes = lax.iota(jnp.int32, 16)
        pltpu.sync_copy(cen_h.at[pl.ds(base, BPW)], cidxa)
        pltpu.sync_copy(ctx_h.at[pl.ds(base, BPW)], xidxa)
        pltpu.sync_copy(neg_h.at[pl.ds(base * K, BPW * K)], nidxa)

        def remap(buf, n):
            def step(i, _):
                sl = pl.ds(i * 16, 16)
                ids = buf[sl]
                buf[sl] = ((ids & jnp.int32(~(2 * CB - 1)))
                           + ((ids & (CB - 1)) << 1)
                           + ((ids >> LOG2CB) & 1))
                return 0
            lax.fori_loop(0, n // 16, step, 0)

        remap(cidxa, BPW)
        remap(xidxa, BPW)
        remap(nidxa, BPW * K)
        bufs = ((cb0, xb0, nb0, sem0), (cb1, xb1, nb1, sem1))

        def descs(t, p):
            cb, xb, nb, sem = bufs[p]
            off = t * C
            return [
                pltpu.make_async_copy(
                    iemb_h.at[cidxa.at[pl.ds(off, C)]], cb, sem),
                pltpu.make_async_copy(
                    oemb_h.at[xidxa.at[pl.ds(off, C)]], xb, sem),
                pltpu.make_async_copy(
                    oemb_h.at[nidxa.at[pl.ds(off * K, C * K)]], nb, sem),
            ]

        def compute(t, p):
            cb, xb, nb, _ = bufs[p]
            for g in range(C // 16):
                rows = g * 16 + lanes
                rows_k = rows * K
                eoff = t * C + g * 16

                def dstep_a(dd, accs):
                    new = list(accs)
                    for u in range(4):
                        col = jnp.broadcast_to(
                            dd * 4 + u, (16,)).astype(jnp.int32)
                        ccol = plsc.load_gather(cb, [rows, col])
                        xcol = plsc.load_gather(xb, [rows, col])
                        new[0] = new[0] + ccol * xcol
                        for k in range(K // 2):
                            ncol = plsc.load_gather(nb, [rows_k + k, col])
                            new[k + 1] = new[k + 1] + ccol * ncol
                    return tuple(new)

                accs = lax.fori_loop(
                    0, D // 4, dstep_a,
                    tuple(jnp.zeros((16,), jnp.float32)
                          for _ in range(1 + K // 2)))
                sbuf[pl.ds(eoff, 16)] = -accs[0]
                for k in range(K // 2):
                    sbuf[pl.ds((k + 1) * BPW + eoff, 16)] = accs[k + 1]

                def dstep_b(dd, accs):
                    new = list(accs)
                    for u in range(4):
                        col = jnp.broadcast_to(
                            dd * 4 + u, (16,)).astype(jnp.int32)
                        ccol = plsc.load_gather(cb, [rows, col])
                        for k in range(K // 2):
                            ncol = plsc.load_gather(
                                nb, [rows_k + (K // 2 + k), col])
                            new[k] = new[k] + ccol * ncol
                    return tuple(new)

                accs = lax.fori_loop(
                    0, D // 4, dstep_b,
                    tuple(jnp.zeros((16,), jnp.float32)
                          for _ in range(K // 2)))
                for k in range(K // 2):
                    sbuf[pl.ds((K // 2 + k + 1) * BPW + eoff, 16)] = accs[k]

        for cp in descs(0, 0):
            cp.start()

        def pair(tt, _):
            for p in range(2):
                t = tt * 2 + p

                @pl.when(t + 1 < NCHUNK)
                def _():
                    for cp in descs(t + 1, 1 - p):
                        cp.start()

                for cp in descs(t, p):
                    cp.wait()
                compute(t, p)
            return 0

        lax.fori_loop(0, NCHUNK // 2, pair, 0)
        pltpu.sync_copy(sbuf, out_h.at[pl.ds(wid * SB, SB)])

    return body(centers, contexts, neg_flat, iemb, oemb)


def _loss_tc(scores):
    def body(s_ref, o_ref):
        x = s_ref[...]
        sp = jnp.maximum(x, 0.0) + jnp.log1p(jnp.exp(-jnp.abs(x)))
        o_ref[0, 0] = jnp.sum(sp) * (1.0 / B)

    return pl.pallas_call(
        body,
        out_shape=jax.ShapeDtypeStruct((1, 1), jnp.float32),
        out_specs=pl.BlockSpec(memory_space=pltpu.SMEM),
    )(scores)


def kernel(input_emb, output_emb, centers, contexts, negatives):
    iemb = _transpose_tc(input_emb.T).reshape(VP, D)
    oemb = _transpose_tc(output_emb.T).reshape(VP, D)
    neg_flat = negatives.astype(jnp.int32).reshape(B * K)
    scores = _sc_scores(centers.astype(jnp.int32), contexts.astype(jnp.int32),
                        neg_flat, iemb, oemb)
    loss = _loss_tc(scores.reshape(NW * (1 + K), BPW))
    return loss[0, 0]


# speedup vs baseline: 1.2622x; 1.0752x over previous
import functools

import jax
import jax.numpy as jnp
from jax import lax
from jax.experimental import pallas as pl
from jax.experimental.pallas import tpu as pltpu
from jax.experimental.pallas import tpu_sc as plsc

B = 16384
D = 64
K = 20
V = 1000000

_info = plsc.get_sparse_core_info()
NC, NS = _info.num_cores, _info.num_subcores
NW = NC * NS
BPW = B // NW
C = 32
NCHUNK = BPW // C
SB = (1 + K) * BPW

CB = 8192
LOG2CB = 13
NPAIR = (V + 2 * CB - 1) // (2 * CB)
VP = NPAIR * 2 * CB


def _tp_body(a_ref, b_ref, o_ref):
    o_ref[...] = jnp.concatenate([a_ref[...].T, b_ref[...].T], axis=1)


def _transpose_tc(embT):
    return pl.pallas_call(
        _tp_body,
        grid=(NPAIR,),
        in_specs=[pl.BlockSpec((D, CB), lambda i: (0, 2 * i)),
                  pl.BlockSpec(
                      (D, CB),
                      lambda i: (0, jnp.minimum(2 * i + 1, (V - 1) // CB)))],
        out_specs=pl.BlockSpec((CB, 2 * D), lambda i: (i, 0)),
        out_shape=jax.ShapeDtypeStruct((NPAIR * CB, 2 * D), jnp.float32),
    )(embT, embT)


def _sc_scores(centers, contexts, neg_flat, iemb, oemb):
    mesh = plsc.VectorSubcoreMesh(core_axis_name="c", subcore_axis_name="s")

    @functools.partial(
        pl.kernel,
        mesh=mesh,
        out_type=jax.ShapeDtypeStruct((NW * SB,), jnp.float32),
        compiler_params=pltpu.CompilerParams(
            needs_layout_passes=False, use_tc_tiling_on_sc=False),
        scratch_types=[
            pltpu.VMEM((BPW,), jnp.int32),
            pltpu.VMEM((BPW,), jnp.int32),
            pltpu.VMEM((BPW * K,), jnp.int32),
            pltpu.VMEM((C, D), jnp.float32),
            pltpu.VMEM((C, D), jnp.float32),
            pltpu.VMEM((C, D), jnp.float32),
            pltpu.VMEM((C, D), jnp.float32),
            pltpu.VMEM((C * K, D), jnp.float32),
            pltpu.VMEM((C * K, D), jnp.float32),
            pltpu.VMEM((SB,), jnp.float32),
            pltpu.SemaphoreType.DMA,
            pltpu.SemaphoreType.DMA,
        ],
    )
    def body(cen_h, ctx_h, neg_h, iemb_h, oemb_h, out_h,
             cidxa, xidxa, nidxa, cb0, cb1, xb0, xb1, nb0, nb1,
             sbuf, sem0, sem1):
        wid = lax.axis_index("s") * NC + lax.axis_index("c")
        base = wid * BPW
        lanes = lax.iota(jnp.int32, 16)
        pltpu.sync_copy(cen_h.at[pl.ds(base, BPW)], cidxa)
        pltpu.sync_copy(ctx_h.at[pl.ds(base, BPW)], xidxa)
        pltpu.sync_copy(neg_h.at[pl.ds(base * K, BPW * K)], nidxa)

        def remap(buf, n):
            def step(i, _):
                sl = pl.ds(i * 16, 16)
                ids = buf[sl]
                buf[sl] = ((ids & jnp.int32(~(2 * CB - 1)))
                           + ((ids & (CB - 1)) << 1)
                           + ((ids >> LOG2CB) & 1))
                return 0
            lax.fori_loop(0, n // 16, step, 0)

        remap(cidxa, BPW)
        remap(xidxa, BPW)
        remap(nidxa, BPW * K)
        bufs = ((cb0, xb0, nb0, sem0), (cb1, xb1, nb1, sem1))

        def descs(t, p):
            cb, xb, nb, sem = bufs[p]
            off = t * C
            return [
                pltpu.make_async_copy(
                    iemb_h.at[cidxa.at[pl.ds(off, C)]], cb, sem),
                pltpu.make_async_copy(
                    oemb_h.at[xidxa.at[pl.ds(off, C)]], xb, sem),
                pltpu.make_async_copy(
                    oemb_h.at[nidxa.at[pl.ds(off * K, C * K)]], nb, sem),
            ]

        def compute(t, p):
            cb, xb, nb, _ = bufs[p]
            for g in range(C // 16):
                rows = g * 16 + lanes
                rows_k = rows * K
                eoff = t * C + g * 16

                def dstep_a(dd, accs):
                    new = list(accs)
                    for u in range(4):
                        col = jnp.broadcast_to(
                            dd * 4 + u, (16,)).astype(jnp.int32)
                        ccol = plsc.load_gather(cb, [rows, col])
                        xcol = plsc.load_gather(xb, [rows, col])
                        new[0] = new[0] + ccol * xcol
                        for k in range(K // 2):
                            ncol = plsc.load_gather(nb, [rows_k + k, col])
                            new[k + 1] = new[k + 1] + ccol * ncol
                    return tuple(new)

                accs = lax.fori_loop(
                    0, D // 4, dstep_a,
                    tuple(jnp.zeros((16,), jnp.float32)
                          for _ in range(1 + K // 2)))
                sbuf[pl.ds(eoff, 16)] = -accs[0]
                for k in range(K // 2):
                    sbuf[pl.ds((k + 1) * BPW + eoff, 16)] = accs[k + 1]

                def dstep_b(dd, accs):
                    new = list(accs)
                    for u in range(4):
                        col = jnp.broadcast_to(
                            dd * 4 + u, (16,)).astype(jnp.int32)
                        ccol = plsc.load_gather(cb, [rows, col])
                        for k in range(K // 2):
                            ncol = plsc.load_gather(
                                nb, [rows_k + (K // 2 + k), col])
                            new[k] = new[k] + ccol * ncol
                    return tuple(new)

                accs = lax.fori_loop(
                    0, D // 4, dstep_b,
                    tuple(jnp.zeros((16,), jnp.float32)
                          for _ in range(K // 2)))
                for k in range(K // 2):
                    sbuf[pl.ds((K // 2 + k + 1) * BPW + eoff, 16)] = accs[k]

        for cp in descs(0, 0):
            cp.start()

        def pair(tt, _):
            for p in range(2):
                t = tt * 2 + p

                @pl.when(t + 1 < NCHUNK)
                def _():
                    for cp in descs(t + 1, 1 - p):
                        cp.start()

                for cp in descs(t, p):
                    cp.wait()
                compute(t, p)
            return 0

        lax.fori_loop(0, NCHUNK // 2, pair, 0)
        pltpu.sync_copy(sbuf, out_h.at[pl.ds(wid * SB, SB)])

    return body(centers, contexts, neg_flat, iemb, oemb)


def _loss_tc(scores):
    def body(s_ref, o_ref):
        x = s_ref[...]
        sp = jnp.maximum(x, 0.0) + jnp.log1p(jnp.exp(-jnp.abs(x)))
        o_ref[0, 0] = jnp.sum(sp) * (1.0 / B)

    return pl.pallas_call(
        body,
        out_shape=jax.ShapeDtypeStruct((1, 1), jnp.float32),
        out_specs=pl.BlockSpec(memory_space=pltpu.SMEM),
    )(scores)


def kernel(input_emb, output_emb, centers, contexts, negatives):
    iemb = _transpose_tc(input_emb.T).reshape(VP, D)
    oemb = _transpose_tc(output_emb.T).reshape(VP, D)
    neg_flat = negatives.astype(jnp.int32).reshape(B * K)
    scores = _sc_scores(centers.astype(jnp.int32), contexts.astype(jnp.int32),
                        neg_flat, iemb, oemb)
    loss = _loss_tc(scores.reshape(NW * (1 + K), BPW))
    return loss[0, 0]


# speedup vs baseline: 1.3068x; 1.0354x over previous
import functools

import jax
import jax.numpy as jnp
from jax import lax
from jax.experimental import pallas as pl
from jax.experimental.pallas import tpu as pltpu
from jax.experimental.pallas import tpu_sc as plsc

B = 16384
D = 64
K = 20
V = 1000000

_info = plsc.get_sparse_core_info()
NC, NS = _info.num_cores, _info.num_subcores
NW = NC * NS
BPW = B // NW
C = 32
NCHUNK = BPW // C
SB = (1 + K) * BPW

CB = 16384
LOG2CB = 14
NPAIR = (V + 2 * CB - 1) // (2 * CB)
VP = NPAIR * 2 * CB


def _tp_body(a_ref, b_ref, o_ref):
    o_ref[...] = jnp.concatenate([a_ref[...].T, b_ref[...].T], axis=1)


def _transpose_tc(embT):
    return pl.pallas_call(
        _tp_body,
        grid=(NPAIR,),
        in_specs=[pl.BlockSpec((D, CB), lambda i: (0, 2 * i)),
                  pl.BlockSpec(
                      (D, CB),
                      lambda i: (0, jnp.minimum(2 * i + 1, (V - 1) // CB)))],
        out_specs=pl.BlockSpec((CB, 2 * D), lambda i: (i, 0)),
        out_shape=jax.ShapeDtypeStruct((NPAIR * CB, 2 * D), jnp.float32),
    )(embT, embT)


def _sc_scores(centers, contexts, neg_flat, iemb, oemb):
    mesh = plsc.VectorSubcoreMesh(core_axis_name="c", subcore_axis_name="s")

    @functools.partial(
        pl.kernel,
        mesh=mesh,
        out_type=jax.ShapeDtypeStruct((NW * SB,), jnp.float32),
        compiler_params=pltpu.CompilerParams(
            needs_layout_passes=False, use_tc_tiling_on_sc=False),
        scratch_types=[
            pltpu.VMEM((BPW,), jnp.int32),
            pltpu.VMEM((BPW,), jnp.int32),
            pltpu.VMEM((BPW * K,), jnp.int32),
            pltpu.VMEM((C, D), jnp.float32),
            pltpu.VMEM((C, D), jnp.float32),
            pltpu.VMEM((C, D), jnp.float32),
            pltpu.VMEM((C, D), jnp.float32),
            pltpu.VMEM((C * K, D), jnp.float32),
            pltpu.VMEM((C * K, D), jnp.float32),
            pltpu.VMEM((SB,), jnp.float32),
            pltpu.SemaphoreType.DMA,
            pltpu.SemaphoreType.DMA,
        ],
    )
    def body(cen_h, ctx_h, neg_h, iemb_h, oemb_h, out_h,
             cidxa, xidxa, nidxa, cb0, cb1, xb0, xb1, nb0, nb1,
             sbuf, sem0, sem1):
        wid = lax.axis_index("s") * NC + lax.axis_index("c")
        base = wid * BPW
        lanes = lax.iota(jnp.int32, 16)
        pltpu.sync_copy(cen_h.at[pl.ds(base, BPW)], cidxa)
        pltpu.sync_copy(ctx_h.at[pl.ds(base, BPW)], xidxa)
        pltpu.sync_copy(neg_h.at[pl.ds(base * K, BPW * K)], nidxa)

        def remap(buf, n):
            def step(i, _):
                sl = pl.ds(i * 16, 16)
                ids = buf[sl]
                buf[sl] = ((ids & jnp.int32(~(2 * CB - 1)))
                           + ((ids & (CB - 1)) << 1)
                           + ((ids >> LOG2CB) & 1))
                return 0
            lax.fori_loop(0, n // 16, step, 0)

        remap(cidxa, BPW)
        remap(xidxa, BPW)
        remap(nidxa, BPW * K)
        bufs = ((cb0, xb0, nb0, sem0), (cb1, xb1, nb1, sem1))

        def descs(t, p):
            cb, xb, nb, sem = bufs[p]
            off = t * C
            return [
                pltpu.make_async_copy(
                    iemb_h.at[cidxa.at[pl.ds(off, C)]], cb, sem),
                pltpu.make_async_copy(
                    oemb_h.at[xidxa.at[pl.ds(off, C)]], xb, sem),
                pltpu.make_async_copy(
                    oemb_h.at[nidxa.at[pl.ds(off * K, C * K)]], nb, sem),
            ]

        def compute(t, p):
            cb, xb, nb, _ = bufs[p]
            for g in range(C // 16):
                rows = g * 16 + lanes
                rows_k = rows * K
                eoff = t * C + g * 16

                def dstep_a(dd, accs):
                    new = list(accs)
                    for u in range(4):
                        col = jnp.broadcast_to(
                            dd * 4 + u, (16,)).astype(jnp.int32)
                        ccol = plsc.load_gather(cb, [rows, col])
                        xcol = plsc.load_gather(xb, [rows, col])
                        new[0] = new[0] + ccol * xcol
                        for k in range(K // 2):
                            ncol = plsc.load_gather(nb, [rows_k + k, col])
                            new[k + 1] = new[k + 1] + ccol * ncol
                    return tuple(new)

                accs = lax.fori_loop(
                    0, D // 4, dstep_a,
                    tuple(jnp.zeros((16,), jnp.float32)
                          for _ in range(1 + K // 2)))
                sbuf[pl.ds(eoff, 16)] = -accs[0]
                for k in range(K // 2):
                    sbuf[pl.ds((k + 1) * BPW + eoff, 16)] = accs[k + 1]

                def dstep_b(dd, accs):
                    new = list(accs)
                    for u in range(4):
                        col = jnp.broadcast_to(
                            dd * 4 + u, (16,)).astype(jnp.int32)
                        ccol = plsc.load_gather(cb, [rows, col])
                        for k in range(K // 2):
                            ncol = plsc.load_gather(
                                nb, [rows_k + (K // 2 + k), col])
                            new[k] = new[k] + ccol * ncol
                    return tuple(new)

                accs = lax.fori_loop(
                    0, D // 4, dstep_b,
                    tuple(jnp.zeros((16,), jnp.float32)
                          for _ in range(K // 2)))
                for k in range(K // 2):
                    sbuf[pl.ds((K // 2 + k + 1) * BPW + eoff, 16)] = accs[k]

        for cp in descs(0, 0):
            cp.start()

        def pair(tt, _):
            for p in range(2):
                t = tt * 2 + p

                @pl.when(t + 1 < NCHUNK)
                def _():
                    for cp in descs(t + 1, 1 - p):
                        cp.start()

                for cp in descs(t, p):
                    cp.wait()
                compute(t, p)
            return 0

        lax.fori_loop(0, NCHUNK // 2, pair, 0)
        pltpu.sync_copy(sbuf, out_h.at[pl.ds(wid * SB, SB)])

    return body(centers, contexts, neg_flat, iemb, oemb)


def _loss_tc(scores):
    def body(s_ref, o_ref):
        x = s_ref[...]
        sp = jnp.maximum(x, 0.0) + jnp.log1p(jnp.exp(-jnp.abs(x)))
        o_ref[0, 0] = jnp.sum(sp) * (1.0 / B)

    return pl.pallas_call(
        body,
        out_shape=jax.ShapeDtypeStruct((1, 1), jnp.float32),
        out_specs=pl.BlockSpec(memory_space=pltpu.SMEM),
    )(scores)


def kernel(input_emb, output_emb, centers, contexts, negatives):
    iemb = _transpose_tc(input_emb.T).reshape(VP, D)
    oemb = _transpose_tc(output_emb.T).reshape(VP, D)
    neg_flat = negatives.astype(jnp.int32).reshape(B * K)
    scores = _sc_scores(centers.astype(jnp.int32), contexts.astype(jnp.int32),
                        neg_flat, iemb, oemb)
    loss = _loss_tc(scores.reshape(NW * (1 + K), BPW))
    return loss[0, 0]
